# baseline (speedup 1.0000x reference)
import jax
import jax.numpy as jnp
from jax import lax
from jax.experimental import pallas as pl
from jax.experimental.pallas import tpu as pltpu
from jax.experimental.pallas import tpu_sc as plsc

L = 2; N = 10000; E = 160000; D = 128; DE = 16; T = 4; FOUT = 32; B = 64; STEPS = 5
TD = T * D
NC = 2; NS = 16; NW = NC * NS
NPT = -(-N // NW)
CE = 64
EPAD = E + 2 * CE


def _sc_agg_body(q_hbm, r_hbm, srcs_hbm, ds_hbm, er_hbm, agg_hbm,
                 er_v, idx_v, dsv, qv, rv, acc, sem):
    wid = lax.axis_index("s") * NC + lax.axis_index("c")
    pltpu.sync_copy(er_hbm.at[wid], er_v)
    er = er_v[pl.ds(0, 16)]
    e0 = er[0]
    e1 = er[1]
    e0a = er[2]
    nch = er[3]

    def chunk_body(c, cur):
        cb = pl.multiple_of(e0a + c * CE, 16)
        pltpu.sync_copy(srcs_hbm.at[pl.ds(cb, CE)], idx_v)
        pltpu.sync_copy(ds_hbm.at[pl.ds(cb, CE)], dsv.at[pl.ds(0, CE)])
        pltpu.sync_copy(r_hbm.at[pl.ds(cb, CE)], rv)
        pltpu.async_copy(q_hbm.at[idx_v], qv, sem).wait()

        def edge_body(j, cur):
            eg = cb + j
            valid = (eg >= e0) & (eg < e1)
            d = dsv[pl.ds(j, 16)][0]
            is_new = valid & (d != cur)

            @pl.when(is_new & (cur >= 0))
            def _():
                pltpu.sync_copy(acc, agg_hbm.at[cur])

            def feat_body(f, carry):
                o = pl.multiple_of(f * 16, 16)
                m = qv[j, pl.ds(o, 16)] + rv[j, pl.ds(o, 16)]
                s_old = acc[pl.ds(o, 16)]
                acc[pl.ds(o, 16)] = jnp.where(
                    is_new, m, jnp.where(valid, s_old + m, s_old))
                n_old = acc[pl.ds(TD + o, 16)]
                acc[pl.ds(TD + o, 16)] = jnp.where(
                    is_new, m, jnp.where(valid, jnp.minimum(n_old, m), n_old))
                x_old = acc[pl.ds(2 * TD + o, 16)]
                acc[pl.ds(2 * TD + o, 16)] = jnp.where(
                    is_new, m, jnp.where(valid, jnp.maximum(x_old, m), x_old))
                return carry

            lax.fori_loop(0, TD // 16, feat_body, jnp.int32(0))
            return jnp.where(is_new, d, cur)

        return lax.fori_loop(0, CE, edge_body, cur)

    cur = lax.fori_loop(0, nch, chunk_body, jnp.int32(-1))

    @pl.when(cur >= 0)
    def _():
        pltpu.sync_copy(acc, agg_hbm.at[cur])


def _sc_agg(q, r, srcs, ds, erange):
    mesh = plsc.VectorSubcoreMesh(core_axis_name="c", subcore_axis_name="s")
    return pl.kernel(
        _sc_agg_body,
        out_type=jax.ShapeDtypeStruct((N, 3 * TD), jnp.float32),
        scratch_types=[
            pltpu.VMEM((16,), jnp.int32),
            pltpu.VMEM((CE,), jnp.int32),
            pltpu.VMEM((CE + 16,), jnp.int32),
            pltpu.VMEM((CE, TD), jnp.float32),
            pltpu.VMEM((CE, TD), jnp.float32),
            pltpu.VMEM((3 * TD,), jnp.float32),
            pltpu.SemaphoreType.DMA,
        ],
        mesh=mesh,
    )(q, r, srcs, ds, erange)


def _lstm_cell(xin, h, c, Wih, Whh, bih, bhh):
    g = xin @ Wih.T + h @ Whh.T + bih + bhh
    i, f, gg, o = jnp.split(g, 4, axis=-1)
    i = jax.nn.sigmoid(i); f = jax.nn.sigmoid(f); gg = jnp.tanh(gg); o = jax.nn.sigmoid(o)
    c2 = f * c + i * gg
    return o * jnp.tanh(c2), c2


def _set2set(x, batch, Wih0, Whh0, bih0, bhh0, Wih1, Whh1, bih1, bhh1):
    d = x.shape[1]
    q_star = jnp.zeros((B, 2 * d), x.dtype)
    h0 = jnp.zeros((B, d), x.dtype); c0 = jnp.zeros((B, d), x.dtype)
    h1 = jnp.zeros((B, d), x.dtype); c1 = jnp.zeros((B, d), x.dtype)
    for _ in range(STEPS):
        h0, c0 = _lstm_cell(q_star, h0, c0, Wih0, Whh0, bih0, bhh0)
        h1, c1 = _lstm_cell(h0, h1, c1, Wih1, Whh1, bih1, bhh1)
        q = h1
        e = jnp.sum(x * q[batch], axis=-1)
        m = jax.ops.segment_max(e, batch, num_segments=B)
        m = jnp.where(jnp.isfinite(m), m, 0.0)
        ex = jnp.exp(e - m[batch])
        s = jax.ops.segment_sum(ex, batch, num_segments=B)
        a = ex / (s[batch] + 1e-16)
        r = jax.ops.segment_sum(a[:, None] * x, batch, num_segments=B)
        q_star = jnp.concatenate([q, r], axis=-1)
    return q_star


def kernel(x, edge_index, edge_attr, batch, We, be, Wpre, bpre, Wpost, bpost,
           Wlin, blin, gamma, beta, Wih0, Whh0, bih0, bhh0, Wih1, Whh1, bih1, bhh1):
    src, dst = edge_index[0], edge_index[1]
    perm = jnp.argsort(dst)
    ds = dst[perm]
    srcs = src[perm]
    eas = edge_attr[perm]

    bounds = jnp.searchsorted(ds, jnp.arange(N + 1), side='left').astype(jnp.int32)
    deg = (bounds[1:] - bounds[:-1]).astype(x.dtype)
    degc = jnp.clip(deg, 1.0)
    has = (deg > 0)[:, None]

    cuts = jnp.minimum(jnp.arange(NW + 1) * NPT, N)
    starts = bounds[cuts]
    e0 = starts[:-1]
    e1 = starts[1:]
    e0a = e0 - jnp.remainder(e0, 16)
    nch = (e1 - e0a + CE - 1) // CE
    erange = jnp.zeros((NW, 16), jnp.int32)
    erange = erange.at[:, 0].set(e0).at[:, 1].set(e1).at[:, 2].set(e0a).at[:, 3].set(nch)

    srcs_pad = jnp.concatenate([srcs, jnp.zeros((EPAD - E,), jnp.int32)])
    ds_pad = jnp.concatenate([ds, jnp.zeros((EPAD - E,), jnp.int32)])
    eas_pad = jnp.concatenate([eas, jnp.zeros((EPAD - E, DE), eas.dtype)])

    folded = []
    for l in range(L):
        WA = Wpre[l, :, :D, :]
        WB = Wpre[l, :, D:2 * D, :]
        WC = Wpre[l, :, 2 * D:, :]
        WAf = WA.transpose(1, 0, 2).reshape(D, TD)
        WBf = WB.transpose(1, 0, 2).reshape(D, TD)
        Wec = jnp.einsum('df,tfg->dtg', We[l], WC).reshape(DE, TD)
        bconst = (jnp.einsum('f,tfg->tg', be[l], WC) + bpre[l]).reshape(TD)
        WlinT = Wlin[l].reshape(T, FOUT, D)
        WXL = jnp.einsum('tfg,tgd->fd', Wpost[l, :, :D, :], WlinT)
        WML = jnp.einsum('tfg,tgd->tfd', Wpost[l, :, D:2 * D, :], WlinT).reshape(TD, D)
        WNL = jnp.einsum('tfg,tgd->tfd', Wpost[l, :, 2 * D:3 * D, :], WlinT).reshape(TD, D)
        WKL = jnp.einsum('tfg,tgd->tfd', Wpost[l, :, 3 * D:, :], WlinT).reshape(TD, D)
        bL = jnp.einsum('tg,tgd->d', bpost[l], WlinT) + blin[l]
        folded.append((WAf, WBf, Wec, bconst, WXL, WML, WNL, WKL, bL))

    h = x
    layer_outs = []
    for l in range(L):
        WAf, WBf, Wec, bconst, WXL, WML, WNL, WKL, bL = folded[l]
        Q = h @ WBf
        base = h @ WAf + bconst
        Rm = eas_pad @ Wec
        agg = _sc_agg(Q, Rm, srcs_pad, ds_pad, erange)
        S = agg[:, :TD]
        MN = agg[:, TD:2 * TD]
        MX = agg[:, 2 * TD:]
        meanf = jnp.where(has, base + S / degc[:, None], 0.0)
        mnf = jnp.where(has, base + MN, 0.0)
        mxf = jnp.where(has, base + MX, 0.0)
        pre = h @ WXL + meanf @ WML + mnf @ WNL + mxf @ WKL + bL
        mu = pre.mean(0)
        var = pre.var(0)
        pre = (pre - mu) / jnp.sqrt(var + 1e-5) * gamma[l] + beta[l]
        h = jax.nn.leaky_relu(pre, 0.01)
        layer_outs.append(h)

    hmax = jnp.maximum(layer_outs[0], layer_outs[1])
    return _set2set(hmax, batch, Wih0, Whh0, bih0, bhh0, Wih1, Whh1, bih1, bhh1)


# speedup vs baseline: 1.1689x; 1.1689x over previous
import jax
import jax.numpy as jnp
from jax import lax
from jax.experimental import pallas as pl
from jax.experimental.pallas import tpu as pltpu
from jax.experimental.pallas import tpu_sc as plsc

L = 2; N = 10000; E = 160000; D = 128; DE = 16; T = 4; FOUT = 32; B = 64; STEPS = 5
TD = T * D
NC = 2; NS = 16; NW = NC * NS
NPT = -(-N // NW)
CE = 64
EPAD = E + 2 * CE


def _sc_agg_body(q_hbm, r_hbm, srcs_hbm, ds_hbm, er_hbm, agg_hbm,
                 er_v, idx_v, dsv, qv, rv, acc, sem):
    wid = lax.axis_index("s") * NC + lax.axis_index("c")
    pltpu.sync_copy(er_hbm.at[wid], er_v)
    er = er_v[pl.ds(0, 16)]
    e0 = er[0]
    e1 = er[1]
    e0a = er[2]
    nch = er[3]

    def chunk_body(c, cur):
        cb = pl.multiple_of(e0a + c * CE, 16)
        pltpu.sync_copy(srcs_hbm.at[pl.ds(cb, CE)], idx_v)
        pltpu.sync_copy(ds_hbm.at[pl.ds(cb, CE)], dsv.at[pl.ds(0, CE)])
        pltpu.sync_copy(r_hbm.at[pl.ds(cb, CE)], rv)
        pltpu.async_copy(q_hbm.at[idx_v], qv, sem).wait()

        def edge_body(j, cur):
            eg = cb + j
            valid = (eg >= e0) & (eg < e1)
            d = dsv[pl.ds(j, 16)][0]
            is_new = valid & (d != cur)

            @pl.when(is_new & (cur >= 0))
            def _():
                pltpu.sync_copy(acc, agg_hbm.at[cur])

            for f in range(TD // 16):
                o = f * 16
                m = qv[j, pl.ds(o, 16)] + rv[j, pl.ds(o, 16)]
                s_old = acc[pl.ds(o, 16)]
                acc[pl.ds(o, 16)] = jnp.where(
                    is_new, m, jnp.where(valid, s_old + m, s_old))
                n_old = acc[pl.ds(TD + o, 16)]
                acc[pl.ds(TD + o, 16)] = jnp.where(
                    is_new, m, jnp.where(valid, jnp.minimum(n_old, m), n_old))
                x_old = acc[pl.ds(2 * TD + o, 16)]
                acc[pl.ds(2 * TD + o, 16)] = jnp.where(
                    is_new, m, jnp.where(valid, jnp.maximum(x_old, m), x_old))
            return jnp.where(is_new, d, cur)

        return lax.fori_loop(0, CE, edge_body, cur)

    cur = lax.fori_loop(0, nch, chunk_body, jnp.int32(-1))

    @pl.when(cur >= 0)
    def _():
        pltpu.sync_copy(acc, agg_hbm.at[cur])


def _sc_agg(q, r, srcs, ds, erange):
    mesh = plsc.VectorSubcoreMesh(core_axis_name="c", subcore_axis_name="s")
    return pl.kernel(
        _sc_agg_body,
        out_type=jax.ShapeDtypeStruct((N, 3 * TD), jnp.float32),
        scratch_types=[
            pltpu.VMEM((16,), jnp.int32),
            pltpu.VMEM((CE,), jnp.int32),
            pltpu.VMEM((CE + 16,), jnp.int32),
            pltpu.VMEM((CE, TD), jnp.float32),
            pltpu.VMEM((CE, TD), jnp.float32),
            pltpu.VMEM((3 * TD,), jnp.float32),
            pltpu.SemaphoreType.DMA,
        ],
        mesh=mesh,
    )(q, r, srcs, ds, erange)


def _lstm_cell(xin, h, c, Wih, Whh, bih, bhh):
    g = xin @ Wih.T + h @ Whh.T + bih + bhh
    i, f, gg, o = jnp.split(g, 4, axis=-1)
    i = jax.nn.sigmoid(i); f = jax.nn.sigmoid(f); gg = jnp.tanh(gg); o = jax.nn.sigmoid(o)
    c2 = f * c + i * gg
    return o * jnp.tanh(c2), c2


def _set2set(x, batch, Wih0, Whh0, bih0, bhh0, Wih1, Whh1, bih1, bhh1):
    d = x.shape[1]
    q_star = jnp.zeros((B, 2 * d), x.dtype)
    h0 = jnp.zeros((B, d), x.dtype); c0 = jnp.zeros((B, d), x.dtype)
    h1 = jnp.zeros((B, d), x.dtype); c1 = jnp.zeros((B, d), x.dtype)
    for _ in range(STEPS):
        h0, c0 = _lstm_cell(q_star, h0, c0, Wih0, Whh0, bih0, bhh0)
        h1, c1 = _lstm_cell(h0, h1, c1, Wih1, Whh1, bih1, bhh1)
        q = h1
        e = jnp.sum(x * q[batch], axis=-1)
        m = jax.ops.segment_max(e, batch, num_segments=B)
        m = jnp.where(jnp.isfinite(m), m, 0.0)
        ex = jnp.exp(e - m[batch])
        s = jax.ops.segment_sum(ex, batch, num_segments=B)
        a = ex / (s[batch] + 1e-16)
        r = jax.ops.segment_sum(a[:, None] * x, batch, num_segments=B)
        q_star = jnp.concatenate([q, r], axis=-1)
    return q_star


def kernel(x, edge_index, edge_attr, batch, We, be, Wpre, bpre, Wpost, bpost,
           Wlin, blin, gamma, beta, Wih0, Whh0, bih0, bhh0, Wih1, Whh1, bih1, bhh1):
    src, dst = edge_index[0], edge_index[1]
    perm = jnp.argsort(dst)
    ds = dst[perm]
    srcs = src[perm]
    eas = edge_attr[perm]

    bounds = jnp.searchsorted(ds, jnp.arange(N + 1), side='left').astype(jnp.int32)
    deg = (bounds[1:] - bounds[:-1]).astype(x.dtype)
    degc = jnp.clip(deg, 1.0)
    has = (deg > 0)[:, None]

    cuts = jnp.minimum(jnp.arange(NW + 1) * NPT, N)
    starts = bounds[cuts]
    e0 = starts[:-1]
    e1 = starts[1:]
    e0a = e0 - jnp.remainder(e0, 16)
    nch = (e1 - e0a + CE - 1) // CE
    erange = jnp.zeros((NW, 16), jnp.int32)
    erange = erange.at[:, 0].set(e0).at[:, 1].set(e1).at[:, 2].set(e0a).at[:, 3].set(nch)

    srcs_pad = jnp.concatenate([srcs, jnp.zeros((EPAD - E,), jnp.int32)])
    ds_pad = jnp.concatenate([ds, jnp.zeros((EPAD - E,), jnp.int32)])
    eas_pad = jnp.concatenate([eas, jnp.zeros((EPAD - E, DE), eas.dtype)])

    folded = []
    for l in range(L):
        WA = Wpre[l, :, :D, :]
        WB = Wpre[l, :, D:2 * D, :]
        WC = Wpre[l, :, 2 * D:, :]
        WAf = WA.transpose(1, 0, 2).reshape(D, TD)
        WBf = WB.transpose(1, 0, 2).reshape(D, TD)
        Wec = jnp.einsum('df,tfg->dtg', We[l], WC).reshape(DE, TD)
        bconst = (jnp.einsum('f,tfg->tg', be[l], WC) + bpre[l]).reshape(TD)
        WlinT = Wlin[l].reshape(T, FOUT, D)
        WXL = jnp.einsum('tfg,tgd->fd', Wpost[l, :, :D, :], WlinT)
        WML = jnp.einsum('tfg,tgd->tfd', Wpost[l, :, D:2 * D, :], WlinT).reshape(TD, D)
        WNL = jnp.einsum('tfg,tgd->tfd', Wpost[l, :, 2 * D:3 * D, :], WlinT).reshape(TD, D)
        WKL = jnp.einsum('tfg,tgd->tfd', Wpost[l, :, 3 * D:, :], WlinT).reshape(TD, D)
        bL = jnp.einsum('tg,tgd->d', bpost[l], WlinT) + blin[l]
        folded.append((WAf, WBf, Wec, bconst, WXL, WML, WNL, WKL, bL))

    h = x
    layer_outs = []
    for l in range(L):
        WAf, WBf, Wec, bconst, WXL, WML, WNL, WKL, bL = folded[l]
        Q = h @ WBf
        base = h @ WAf + bconst
        Rm = eas_pad @ Wec
        agg = _sc_agg(Q, Rm, srcs_pad, ds_pad, erange)
        S = agg[:, :TD]
        MN = agg[:, TD:2 * TD]
        MX = agg[:, 2 * TD:]
        meanf = jnp.where(has, base + S / degc[:, None], 0.0)
        mnf = jnp.where(has, base + MN, 0.0)
        mxf = jnp.where(has, base + MX, 0.0)
        pre = h @ WXL + meanf @ WML + mnf @ WNL + mxf @ WKL + bL
        mu = pre.mean(0)
        var = pre.var(0)
        pre = (pre - mu) / jnp.sqrt(var + 1e-5) * gamma[l] + beta[l]
        h = jax.nn.leaky_relu(pre, 0.01)
        layer_outs.append(h)

    hmax = jnp.maximum(layer_outs[0], layer_outs[1])
    return _set2set(hmax, batch, Wih0, Whh0, bih0, bhh0, Wih1, Whh1, bih1, bhh1)


# speedup vs baseline: 1.3803x; 1.1808x over previous
import functools

import jax
import jax.numpy as jnp
from jax import lax
from jax.experimental import pallas as pl
from jax.experimental.pallas import tpu as pltpu
from jax.experimental.pallas import tpu_sc as plsc

L = 2; N = 10000; E = 160000; D = 128; DE = 16; T = 4; FOUT = 32; B = 64; STEPS = 5
TD = T * D
NC = 2; NS = 16; NW = NC * NS
NPT = -(-N // NW)
CE = 64
EPAD = 161792


def _sc_agg_body(q_hbm, r_hbm, srcs_hbm, ds_hbm, er_hbm, agg_hbm,
                 er_v, idx_v, dsv, qv, rv, acc, sem):
    wid = lax.axis_index("s") * NC + lax.axis_index("c")
    pltpu.sync_copy(er_hbm.at[wid], er_v)
    er = er_v[pl.ds(0, 16)]
    e0 = er[0]
    e1 = er[1]
    e0a = er[2]
    nch = er[3]

    def chunk_body(c, cur):
        cb = pl.multiple_of(e0a + c * CE, 16)
        pltpu.sync_copy(srcs_hbm.at[pl.ds(cb, CE)], idx_v)
        pltpu.sync_copy(ds_hbm.at[pl.ds(cb, CE)], dsv.at[pl.ds(0, CE)])
        pltpu.sync_copy(r_hbm.at[pl.ds(cb, CE)], rv)
        pltpu.async_copy(q_hbm.at[idx_v], qv, sem).wait()

        def edge_body(j, cur):
            eg = cb + j
            valid = (eg >= e0) & (eg < e1)
            d = dsv[pl.ds(j, 16)][0]
            is_new = valid & (d != cur)

            @pl.when(is_new & (cur >= 0))
            def _():
                pltpu.sync_copy(acc, agg_hbm.at[cur])

            for f in range(TD // 16):
                o = f * 16
                m = qv[j, pl.ds(o, 16)] + rv[j, pl.ds(o, 16)]
                s_old = acc[pl.ds(o, 16)]
                acc[pl.ds(o, 16)] = jnp.where(
                    is_new, m, jnp.where(valid, s_old + m, s_old))
                n_old = acc[pl.ds(TD + o, 16)]
                acc[pl.ds(TD + o, 16)] = jnp.where(
                    is_new, m, jnp.where(valid, jnp.minimum(n_old, m), n_old))
                x_old = acc[pl.ds(2 * TD + o, 16)]
                acc[pl.ds(2 * TD + o, 16)] = jnp.where(
                    is_new, m, jnp.where(valid, jnp.maximum(x_old, m), x_old))
            return jnp.where(is_new, d, cur)

        return lax.fori_loop(0, CE, edge_body, cur)

    cur = lax.fori_loop(0, nch, chunk_body, jnp.int32(-1))

    @pl.when(cur >= 0)
    def _():
        pltpu.sync_copy(acc, agg_hbm.at[cur])


def _sc_agg(q, r, srcs, ds, erange):
    mesh = plsc.VectorSubcoreMesh(core_axis_name="c", subcore_axis_name="s")
    return pl.kernel(
        _sc_agg_body,
        out_type=jax.ShapeDtypeStruct((N, 3 * TD), jnp.float32),
        scratch_types=[
            pltpu.VMEM((16,), jnp.int32),
            pltpu.VMEM((CE,), jnp.int32),
            pltpu.VMEM((CE + 16,), jnp.int32),
            pltpu.VMEM((CE, TD), jnp.float32),
            pltpu.VMEM((CE, TD), jnp.float32),
            pltpu.VMEM((3 * TD,), jnp.float32),
            pltpu.SemaphoreType.DMA,
        ],
        mesh=mesh,
    )(q, r, srcs, ds, erange)


BN1 = 2000
BNR = 2048
BN2 = 1000


def _k1_body(h_ref, w_ref, b_ref, q_ref, base_ref):
    blk = jnp.dot(h_ref[...], w_ref[...], preferred_element_type=jnp.float32) + b_ref[...]
    q_ref[...] = blk[:, :TD]
    base_ref[...] = blk[:, TD:]


def _k1(h, w, bcat):
    return pl.pallas_call(
        _k1_body,
        grid=(N // BN1,),
        in_specs=[
            pl.BlockSpec((BN1, D), lambda i: (i, 0)),
            pl.BlockSpec((D, 2 * TD), lambda i: (0, 0)),
            pl.BlockSpec((1, 2 * TD), lambda i: (0, 0)),
        ],
        out_specs=[
            pl.BlockSpec((BN1, TD), lambda i: (i, 0)),
            pl.BlockSpec((BN1, TD), lambda i: (i, 0)),
        ],
        out_shape=[
            jax.ShapeDtypeStruct((N, TD), jnp.float32),
            jax.ShapeDtypeStruct((N, TD), jnp.float32),
        ],
    )(h, w, bcat)


def _kr_body(ea_ref, w_ref, r0_ref, r1_ref):
    p = jnp.dot(ea_ref[...], w_ref[...], preferred_element_type=jnp.float32)
    r0_ref[...] = p[:, :TD]
    r1_ref[...] = p[:, TD:]


def _kr(eas_pad, wec2):
    epad = eas_pad.shape[0]
    return pl.pallas_call(
        _kr_body,
        grid=(epad // BNR,),
        in_specs=[
            pl.BlockSpec((BNR, DE), lambda i: (i, 0)),
            pl.BlockSpec((DE, 2 * TD), lambda i: (0, 0)),
        ],
        out_specs=[
            pl.BlockSpec((BNR, TD), lambda i: (i, 0)),
            pl.BlockSpec((BNR, TD), lambda i: (i, 0)),
        ],
        out_shape=[
            jax.ShapeDtypeStruct((epad, TD), jnp.float32),
            jax.ShapeDtypeStruct((epad, TD), jnp.float32),
        ],
    )(eas_pad, wec2)


def _k2a_body(h_ref, base_ref, agg_ref, aux_ref, w_ref, bl_ref, pre_ref, st_ref):
    i = pl.program_id(0)
    agg = agg_ref[...]
    b = base_ref[...]
    iv = aux_ref[:, 0:1]
    hs = aux_ref[:, 1:2] > 0.5
    meanf = jnp.where(hs, b + agg[:, :TD] * iv, 0.0)
    mnf = jnp.where(hs, b + agg[:, TD:2 * TD], 0.0)
    mxf = jnp.where(hs, b + agg[:, 2 * TD:], 0.0)
    v = jnp.concatenate([h_ref[...], meanf, mnf, mxf], axis=1)
    pre = jnp.dot(v, w_ref[...], preferred_element_type=jnp.float32) + bl_ref[...]
    pre_ref[...] = pre

    @pl.when(i == 0)
    def _():
        st_ref[...] = jnp.zeros_like(st_ref)

    st = st_ref[...]
    upd = jnp.concatenate([
        jnp.sum(pre, axis=0, keepdims=True),
        jnp.sum(pre * pre, axis=0, keepdims=True),
        jnp.zeros((6, D), jnp.float32),
    ], axis=0)
    st_ref[...] = st + upd


def _k2a(h, base, agg, aux, wbig, bl):
    return pl.pallas_call(
        _k2a_body,
        grid=(N // BN2,),
        in_specs=[
            pl.BlockSpec((BN2, D), lambda i: (i, 0)),
            pl.BlockSpec((BN2, TD), lambda i: (i, 0)),
            pl.BlockSpec((BN2, 3 * TD), lambda i: (i, 0)),
            pl.BlockSpec((BN2, D), lambda i: (i, 0)),
            pl.BlockSpec((D + 3 * TD, D), lambda i: (0, 0)),
            pl.BlockSpec((1, D), lambda i: (0, 0)),
        ],
        out_specs=[
            pl.BlockSpec((BN2, D), lambda i: (i, 0)),
            pl.BlockSpec((8, D), lambda i: (0, 0)),
        ],
        out_shape=[
            jax.ShapeDtypeStruct((N, D), jnp.float32),
            jax.ShapeDtypeStruct((8, D), jnp.float32),
        ],
    )(h, base, agg, aux, wbig, bl)


def _k2b_body(pre_ref, sc_ref, sh_ref, hprev_ref, out_ref, take_max):
    t = pre_ref[...] * sc_ref[...] + sh_ref[...]
    hn = jnp.where(t > 0, t, 0.01 * t)
    if take_max:
        hn = jnp.maximum(hn, hprev_ref[...])
    out_ref[...] = hn


def _k2b(pre, scale, shift, hprev, take_max):
    return pl.pallas_call(
        functools.partial(_k2b_body, take_max=take_max),
        grid=(N // BN1,),
        in_specs=[
            pl.BlockSpec((BN1, D), lambda i: (i, 0)),
            pl.BlockSpec((1, D), lambda i: (0, 0)),
            pl.BlockSpec((1, D), lambda i: (0, 0)),
            pl.BlockSpec((BN1, D), lambda i: (i, 0)),
        ],
        out_specs=pl.BlockSpec((BN1, D), lambda i: (i, 0)),
        out_shape=jax.ShapeDtypeStruct((N, D), jnp.float32),
    )(pre, scale, shift, hprev)


def _k3_body(x_ref, oh_ref, oht_ref,
             wih0_ref, whh0_ref, b0_ref, wih1_ref, whh1_ref, b1_ref, out_ref):
    x = x_ref[...]
    oh = oh_ref[...]
    oht = oht_ref[...]
    wih0 = wih0_ref[...]; whh0 = whh0_ref[...]; b0 = b0_ref[...]
    wih1 = wih1_ref[...]; whh1 = whh1_ref[...]; b1 = b1_ref[...]
    q_star = jnp.zeros((B, 2 * D), jnp.float32)
    h0 = jnp.zeros((B, D), jnp.float32); c0 = jnp.zeros((B, D), jnp.float32)
    h1 = jnp.zeros((B, D), jnp.float32); c1 = jnp.zeros((B, D), jnp.float32)
    for _ in range(STEPS):
        g = jnp.dot(q_star, wih0, preferred_element_type=jnp.float32) \
            + jnp.dot(h0, whh0, preferred_element_type=jnp.float32) + b0
        ii = jax.nn.sigmoid(g[:, :D]); ff = jax.nn.sigmoid(g[:, D:2 * D])
        gg = jnp.tanh(g[:, 2 * D:3 * D]); oo = jax.nn.sigmoid(g[:, 3 * D:])
        c0 = ff * c0 + ii * gg
        h0 = oo * jnp.tanh(c0)
        g = jnp.dot(h0, wih1, preferred_element_type=jnp.float32) \
            + jnp.dot(h1, whh1, preferred_element_type=jnp.float32) + b1
        ii = jax.nn.sigmoid(g[:, :D]); ff = jax.nn.sigmoid(g[:, D:2 * D])
        gg = jnp.tanh(g[:, 2 * D:3 * D]); oo = jax.nn.sigmoid(g[:, 3 * D:])
        c1 = ff * c1 + ii * gg
        h1 = oo * jnp.tanh(c1)
        q = h1
        tmp = jnp.dot(oh, q, preferred_element_type=jnp.float32)
        e = jnp.sum(x * tmp, axis=1, keepdims=True)
        masked = jnp.where(oh > 0.5, e, -1e30)
        m64 = jnp.max(masked, axis=0, keepdims=True)
        m64 = jnp.where(m64 < -1e29, 0.0, m64)
        mb = jnp.sum(oh * m64, axis=1, keepdims=True)
        ex = jnp.exp(e - mb)
        s64 = jnp.sum(oh * ex, axis=0, keepdims=True)
        sb = jnp.sum(oh * s64, axis=1, keepdims=True)
        a = ex / (sb + 1e-16)
        r = jnp.dot(oht, a * x, preferred_element_type=jnp.float32)
        q_star = jnp.concatenate([q, r], axis=1)
    out_ref[...] = q_star


def _k3(x, oh, oht, wih0t, whh0t, b0, wih1t, whh1t, b1):
    full = lambda shape: pl.BlockSpec(shape, lambda: (0, 0))
    return pl.pallas_call(
        _k3_body,
        grid=(),
        in_specs=[
            full((N, D)), full((N, B)), full((B, N)),
            full((2 * D, 4 * D)), full((D, 4 * D)), full((1, 4 * D)),
            full((D, 4 * D)), full((D, 4 * D)), full((1, 4 * D)),
        ],
        out_specs=full((B, 2 * D)),
        out_shape=jax.ShapeDtypeStruct((B, 2 * D), jnp.float32),
    )(x, oh, oht, wih0t, whh0t, b0, wih1t, whh1t, b1)


def kernel(x, edge_index, edge_attr, batch, We, be, Wpre, bpre, Wpost, bpost,
           Wlin, blin, gamma, beta, Wih0, Whh0, bih0, bhh0, Wih1, Whh1, bih1, bhh1):
    src, dst = edge_index[0], edge_index[1]
    perm = jnp.argsort(dst)
    ds = dst[perm]
    srcs = src[perm]
    eas = edge_attr[perm]

    bounds = jnp.searchsorted(ds, jnp.arange(N + 1), side='left').astype(jnp.int32)
    deg = (bounds[1:] - bounds[:-1]).astype(x.dtype)
    degc = jnp.clip(deg, 1.0)
    has = (deg > 0)[:, None]

    cuts = jnp.minimum(jnp.arange(NW + 1) * NPT, N)
    starts = bounds[cuts]
    e0 = starts[:-1]
    e1 = starts[1:]
    e0a = e0 - jnp.remainder(e0, 16)
    nch = (e1 - e0a + CE - 1) // CE
    erange = jnp.zeros((NW, 16), jnp.int32)
    erange = erange.at[:, 0].set(e0).at[:, 1].set(e1).at[:, 2].set(e0a).at[:, 3].set(nch)

    srcs_pad = jnp.concatenate([srcs, jnp.zeros((EPAD - E,), jnp.int32)])
    ds_pad = jnp.concatenate([ds, jnp.zeros((EPAD - E,), jnp.int32)])
    eas_pad = jnp.concatenate([eas, jnp.zeros((EPAD - E, DE), eas.dtype)])

    iv = jnp.where(deg > 0, 1.0 / degc, 0.0)
    aux = jnp.zeros((N, D), jnp.float32)
    aux = aux.at[:, 0].set(iv).at[:, 1].set((deg > 0).astype(jnp.float32))

    folded = []
    wecs = []
    for l in range(L):
        WA = Wpre[l, :, :D, :]
        WB = Wpre[l, :, D:2 * D, :]
        WC = Wpre[l, :, 2 * D:, :]
        WAf = WA.transpose(1, 0, 2).reshape(D, TD)
        WBf = WB.transpose(1, 0, 2).reshape(D, TD)
        Wec = jnp.einsum('df,tfg->dtg', We[l], WC).reshape(DE, TD)
        bconst = (jnp.einsum('f,tfg->tg', be[l], WC) + bpre[l]).reshape(TD)
        WlinT = Wlin[l].reshape(T, FOUT, D)
        WXL = jnp.einsum('tfg,tgd->fd', Wpost[l, :, :D, :], WlinT)
        WML = jnp.einsum('tfg,tgd->tfd', Wpost[l, :, D:2 * D, :], WlinT).reshape(TD, D)
        WNL = jnp.einsum('tfg,tgd->tfd', Wpost[l, :, 2 * D:3 * D, :], WlinT).reshape(TD, D)
        WKL = jnp.einsum('tfg,tgd->tfd', Wpost[l, :, 3 * D:, :], WlinT).reshape(TD, D)
        bL = (jnp.einsum('tg,tgd->d', bpost[l], WlinT) + blin[l])[None]
        w1 = jnp.concatenate([WBf, WAf], axis=1)
        bcat = jnp.concatenate([jnp.zeros((TD,)), bconst])[None]
        wbig = jnp.concatenate([WXL, WML, WNL, WKL], axis=0)
        folded.append((w1, bcat, wbig, bL))
        wecs.append(Wec)

    wec2 = jnp.concatenate(wecs, axis=1)
    R0, R1 = _kr(eas_pad, wec2)
    Rms = [R0, R1]

    h = x
    h1 = x
    for l in range(L):
        w1, bcat, wbig, bL = folded[l]
        Q, base = _k1(h, w1, bcat)
        agg = _sc_agg(Q, Rms[l], srcs_pad, ds_pad, erange)
        pre, stats = _k2a(h, base, agg, aux, wbig, bL)
        mu = stats[0:1] / N
        var = stats[1:2] / N - mu * mu
        scale = gamma[l][None] / jnp.sqrt(var + 1e-5)
        shift = beta[l][None] - mu * scale
        if l == 0:
            h = _k2b(pre, scale, shift, pre, False)
            h1 = h
        else:
            h = _k2b(pre, scale, shift, h1, True)

    oh = (batch[:, None] == jnp.arange(B)[None, :]).astype(jnp.float32)
    oht = (batch[None, :] == jnp.arange(B)[:, None]).astype(jnp.float32)
    q_star = _k3(h, oh, oht,
                 Wih0.T, Whh0.T, (bih0 + bhh0)[None],
                 Wih1.T, Whh1.T, (bih1 + bhh1)[None])
    return q_star
